# baseline (speedup 1.0000x reference)
import functools

import jax
import jax.numpy as jnp
from jax import lax
from jax.experimental import pallas as pl
from jax.experimental.pallas import tpu as pltpu
from jax.experimental.pallas import tpu_sc as plsc

_NC = 2
_NS = 16
_L = 16
_K = 128

_SPLAT_DNUMS = lax.GatherDimensionNumbers(
    offset_dims=(), collapsed_slice_dims=(0,), start_index_map=(0,))


_G = 16
_NB = 2
_SUB = _K // _L


def _sc_spmm(x, idx1, val1, idx2, val2):
    n, d = x.shape
    chunks_total = idx1.shape[0]
    pt = (chunks_total // _NS) * _K
    groups = chunks_total // (_NS * _G)
    npt = -(-n // (_NS * _K)) * _K
    n_pad = npt * _NS
    assert chunks_total == groups * _NS * _G and d % _L == 0

    mesh = plsc.VectorSubcoreMesh(core_axis_name="c", subcore_axis_name="s")

    @functools.partial(
        pl.kernel,
        mesh=mesh,
        out_type=(
            jax.ShapeDtypeStruct((n_pad, d), jnp.float32),
            jax.ShapeDtypeStruct((n_pad, d), jnp.float32),
        ),
        scratch_types=[
            pltpu.VMEM((_G, 2, _K), jnp.int32),
            pltpu.VMEM((_G, _K), jnp.float32),
            pltpu.VMEM((_K, d), jnp.float32),
            pltpu.SemaphoreType.DMA,
            pltpu.SemaphoreType.DMA,
            pltpu.VMEM_SHARED((n_pad, d), jnp.float32),
        ],
    )
    def spmm_kernel(x_hbm, i1, v1, i2, v2, g1, g2,
                    idx_v, val_v, rows, gsem, ssem, acc):
        cid = lax.axis_index("c")
        tid = lax.axis_index("s")

        zero = jnp.zeros((_L,), jnp.float32)

        @pl.loop(0, _K)
        def _(r):
            for j in range(d // _L):
                rows[r, pl.ds(j * _L, _L)] = zero

        def scale(j):
            @pl.loop(0, _K // _L)
            def _(gg):
                gbase = gg * _L
                val16 = val_v[j, pl.ds(gbase, _L)]
                for l in range(_L):
                    vs = lax.gather(
                        val16, jnp.full((_L, 1), l, jnp.int32),
                        _SPLAT_DNUMS, (1,),
                        mode=lax.GatherScatterMode.PROMISE_IN_BOUNDS)
                    for jj in range(d // _L):
                        sl = pl.ds(jj * _L, _L)
                        rows[gbase + l, sl] = rows[gbase + l, sl] * vs

        def process(i_hbm, v_hbm, out_hbm):
            row0 = tid * npt
            for i in range(npt // _K):
                pltpu.sync_copy(rows, acc.at[pl.ds(row0 + i * _K, _K)])
            plsc.subcore_barrier()

            gchunk0 = tid * (groups * _G)

            @pl.loop(0, groups)
            def _(g):
                row = pl.ds(gchunk0 + g * _G, _G)
                pltpu.sync_copy(i_hbm.at[row], idx_v)
                pltpu.sync_copy(v_hbm.at[row], val_v)

                @pl.loop(0, _G)
                def _(j):
                    pltpu.async_copy(
                        x_hbm.at[idx_v.at[j, 0]], rows, gsem).wait()
                    scale(j)
                    pltpu.async_copy(
                        rows, acc.at[idx_v.at[j, 1]], ssem, add=True).wait()

            plsc.subcore_barrier()
            for i in range(npt // _K):
                pltpu.sync_copy(acc.at[pl.ds(row0 + i * _K, _K)], rows)
                pltpu.sync_copy(rows, out_hbm.at[pl.ds(row0 + i * _K, _K)])

        @pl.when(cid == 0)
        def _():
            process(i1, v1, g1)

        @pl.when(cid == 1)
        def _():
            process(i2, v2, g2)

    return spmm_kernel(x, idx1, val1, idx2, val2)


def _ln_body(x_ref, g1_ref, g2_ref, w_ref, scale_ref, bias_ref, out_ref):
    w = w_ref[...]
    h0 = jnp.dot(x_ref[...], w, preferred_element_type=jnp.float32)
    h1 = jnp.dot(g1_ref[...], w, preferred_element_type=jnp.float32)
    h2 = jnp.dot(g2_ref[...], w, preferred_element_type=jnp.float32)
    h = jnp.concatenate([h0, h1, h2], axis=1)
    mu = jnp.mean(h, axis=1, keepdims=True)
    var = jnp.mean(h * h, axis=1, keepdims=True) - mu * mu
    inv = lax.rsqrt(var + 1e-5)
    out_ref[...] = (h - mu) * inv * scale_ref[...] + bias_ref[...]


def _ln_tc(x, g1, g2, w, ln_scale, ln_bias):
    n, d = x.shape
    out_dim = ln_scale.shape[0]
    bt = 1000
    grid = n // bt
    return pl.pallas_call(
        _ln_body,
        grid=(grid,),
        in_specs=[
            pl.BlockSpec((bt, d), lambda i: (i, 0)),
            pl.BlockSpec((bt, d), lambda i: (i, 0)),
            pl.BlockSpec((bt, d), lambda i: (i, 0)),
            pl.BlockSpec((d, d), lambda i: (0, 0)),
            pl.BlockSpec((1, out_dim), lambda i: (0, 0)),
            pl.BlockSpec((1, out_dim), lambda i: (0, 0)),
        ],
        out_specs=pl.BlockSpec((bt, out_dim), lambda i: (i, 0)),
        out_shape=jax.ShapeDtypeStruct((n, out_dim), jnp.float32),
    )(x, g1, g2, w, ln_scale.reshape(1, -1), ln_bias.reshape(1, -1))


def kernel(x, adj1_indices, adj1_values, adj2_indices, adj2_values,
           W, ln_scale, ln_bias):
    e = adj1_values.shape[0]
    groups = -(-e // (_NS * _G * _K))
    chunks_total = groups * _NS * _G
    epad = chunks_total * _K
    pad = epad - e

    def prep(indices, values):
        dst = indices[0]
        src = indices[1]
        if pad:
            zi = jnp.zeros((pad,), jnp.int32)
            src = jnp.concatenate([src, zi])
            dst = jnp.concatenate([dst, zi])
            values = jnp.concatenate([values, jnp.zeros((pad,), jnp.float32)])
        idx = jnp.stack([src.reshape(chunks_total, _K),
                         dst.reshape(chunks_total, _K)], axis=1)
        return idx, values.reshape(chunks_total, _K)

    i1, v1 = prep(adj1_indices, adj1_values)
    i2, v2 = prep(adj2_indices, adj2_values)
    g1, g2 = _sc_spmm(x, i1, v1, i2, v2)
    return _ln_tc(x, g1, g2, W, ln_scale, ln_bias)


# speedup vs baseline: 1.3835x; 1.3835x over previous
import functools

import jax
import jax.numpy as jnp
from jax import lax
from jax.experimental import pallas as pl
from jax.experimental.pallas import tpu as pltpu
from jax.experimental.pallas import tpu_sc as plsc

_NC = 2
_NS = 16
_L = 16
_K = 128

_SPLAT_DNUMS = lax.GatherDimensionNumbers(
    offset_dims=(), collapsed_slice_dims=(0,), start_index_map=(0,))


def _sc_spmm(x, idx1, val1, idx2, val2):
    n, d = x.shape
    chunks_total = idx1.shape[0]
    cpt = chunks_total // _NS
    npt = -(-n // (_NS * _K)) * _K
    n_pad = npt * _NS
    assert chunks_total == cpt * _NS and d % _L == 0

    mesh = plsc.VectorSubcoreMesh(core_axis_name="c", subcore_axis_name="s")

    @functools.partial(
        pl.kernel,
        mesh=mesh,
        out_type=(
            jax.ShapeDtypeStruct((n_pad, d), jnp.float32),
            jax.ShapeDtypeStruct((n_pad, d), jnp.float32),
        ),
        scratch_types=[
            pltpu.VMEM((2, _K), jnp.int32),
            pltpu.VMEM((1, _K), jnp.float32),
            pltpu.VMEM((_K, d), jnp.float32),
            pltpu.SemaphoreType.DMA,
            pltpu.SemaphoreType.DMA,
            pltpu.VMEM_SHARED((n_pad, d), jnp.float32),
        ],
    )
    def spmm_kernel(x_hbm, i1, v1, i2, v2, g1, g2,
                    idx_v, val_v, rows, gsem, ssem, acc):
        cid = lax.axis_index("c")
        tid = lax.axis_index("s")

        zero = jnp.zeros((_L,), jnp.float32)

        @pl.loop(0, _K)
        def _(r):
            for j in range(d // _L):
                rows[r, pl.ds(j * _L, _L)] = zero

        def scale():
            @pl.loop(0, _K // _L)
            def _(gg):
                gbase = gg * _L
                val16 = val_v[0, pl.ds(gbase, _L)]
                for l in range(_L):
                    vs = lax.gather(
                        val16, jnp.full((_L, 1), l, jnp.int32),
                        _SPLAT_DNUMS, (1,),
                        mode=lax.GatherScatterMode.PROMISE_IN_BOUNDS)
                    for jj in range(d // _L):
                        sl = pl.ds(jj * _L, _L)
                        rows[gbase + l, sl] = rows[gbase + l, sl] * vs

        def process(i_hbm, v_hbm, out_hbm):
            row0 = tid * npt
            for i in range(npt // _K):
                pltpu.sync_copy(rows, acc.at[pl.ds(row0 + i * _K, _K)])
            plsc.subcore_barrier()

            chunk0 = tid * cpt

            @pl.loop(0, cpt)
            def _(c):
                pltpu.sync_copy(i_hbm.at[chunk0 + c], idx_v)
                pltpu.sync_copy(v_hbm.at[pl.ds(chunk0 + c, 1)], val_v)
                pltpu.async_copy(
                    x_hbm.at[idx_v.at[0]], rows, gsem).wait()
                scale()
                pltpu.async_copy(
                    rows, acc.at[idx_v.at[1]], ssem, add=True).wait()

            plsc.subcore_barrier()
            for i in range(npt // _K):
                pltpu.sync_copy(acc.at[pl.ds(row0 + i * _K, _K)], rows)
                pltpu.sync_copy(rows, out_hbm.at[pl.ds(row0 + i * _K, _K)])

        @pl.when(cid == 0)
        def _():
            process(i1, v1, g1)

        @pl.when(cid == 1)
        def _():
            process(i2, v2, g2)

    return spmm_kernel(x, idx1, val1, idx2, val2)


def _ln_body(x_ref, g1_ref, g2_ref, w_ref, scale_ref, bias_ref, out_ref):
    w = w_ref[...]
    h0 = jnp.dot(x_ref[...], w, preferred_element_type=jnp.float32)
    h1 = jnp.dot(g1_ref[...], w, preferred_element_type=jnp.float32)
    h2 = jnp.dot(g2_ref[...], w, preferred_element_type=jnp.float32)
    h = jnp.concatenate([h0, h1, h2], axis=1)
    mu = jnp.mean(h, axis=1, keepdims=True)
    var = jnp.mean(h * h, axis=1, keepdims=True) - mu * mu
    inv = lax.rsqrt(var + 1e-5)
    out_ref[...] = (h - mu) * inv * scale_ref[...] + bias_ref[...]


def _ln_tc(x, g1, g2, w, ln_scale, ln_bias):
    n, d = x.shape
    out_dim = ln_scale.shape[0]
    bt = 1000
    grid = n // bt
    return pl.pallas_call(
        _ln_body,
        grid=(grid,),
        in_specs=[
            pl.BlockSpec((bt, d), lambda i: (i, 0)),
            pl.BlockSpec((bt, d), lambda i: (i, 0)),
            pl.BlockSpec((bt, d), lambda i: (i, 0)),
            pl.BlockSpec((d, d), lambda i: (0, 0)),
            pl.BlockSpec((1, out_dim), lambda i: (0, 0)),
            pl.BlockSpec((1, out_dim), lambda i: (0, 0)),
        ],
        out_specs=pl.BlockSpec((bt, out_dim), lambda i: (i, 0)),
        out_shape=jax.ShapeDtypeStruct((n, out_dim), jnp.float32),
    )(x, g1, g2, w, ln_scale.reshape(1, -1), ln_bias.reshape(1, -1))


def kernel(x, adj1_indices, adj1_values, adj2_indices, adj2_values,
           W, ln_scale, ln_bias):
    e = adj1_values.shape[0]
    chunks_total = -(-e // (_NS * _K)) * _NS
    epad = chunks_total * _K
    pad = epad - e

    def prep(indices, values):
        dst = indices[0]
        src = indices[1]
        if pad:
            zi = jnp.zeros((pad,), jnp.int32)
            src = jnp.concatenate([src, zi])
            dst = jnp.concatenate([dst, zi])
            values = jnp.concatenate([values, jnp.zeros((pad,), jnp.float32)])
        idx = jnp.stack([src.reshape(chunks_total, _K),
                         dst.reshape(chunks_total, _K)], axis=1)
        return idx, values.reshape(chunks_total, _K)

    i1, v1 = prep(adj1_indices, adj1_values)
    i2, v2 = prep(adj2_indices, adj2_values)
    g1, g2 = _sc_spmm(x, i1, v1, i2, v2)
    return _ln_tc(x, g1, g2, W, ln_scale, ln_bias)
